# baseline (speedup 1.0000x reference)
import functools

import jax
import jax.numpy as jnp
from jax import lax
from jax.experimental import pallas as pl
from jax.experimental.pallas import tpu as pltpu
from jax.experimental.pallas import tpu_sc as plsc

N = 10000
E = 320000
D = 128

NC, NS, L = 2, 16, 16
NW = NC * NS
NPAD = 10240
CH = 128
NCH0 = 152
NCH1 = 8
NCHUNK = NCH0 + NCH1
PER_S = NCHUNK * CH
EPAD = PER_S * NS
NCH_TOT = EPAD // CH
ROWS_PER_S = NPAD // NS

_mesh = plsc.VectorSubcoreMesh(core_axis_name="c", subcore_axis_name="s")


def _spmm_body(sup, eidx, out, idx0, idx1, rows0, rows1, acc,
               gsem0, gsem1, ssem0, ssem1):
    c = lax.axis_index("c")
    s = lax.axis_index("s")
    idx = (idx0, idx1)
    rows = (rows0, rows1)
    gsem = (gsem0, gsem1)
    ssem = (ssem0, ssem1)

    zeros = jnp.zeros((L,), jnp.float32)

    def zbody(r, carry):
        for k in range(D // L):
            rows0[r, pl.ds(k * L, L)] = zeros
        return carry

    lax.fori_loop(0, CH, zbody, 0)
    rbase = s * ROWS_PER_S
    for k in range(ROWS_PER_S // CH):
        pltpu.sync_copy(rows0, acc.at[pl.ds(rbase + k * CH, CH)])
    plsc.subcore_barrier()

    cbase = s * NCHUNK + c * NCH0

    def load_idx(i, b):
        pltpu.sync_copy(eidx.at[cbase + i], idx[b])

    def gstart(b):
        pltpu.async_copy(sup.at[idx[b].at[0]], rows[b], gsem[b])

    def sstart(b):
        pltpu.async_copy(rows[b], acc.at[idx[b].at[1]], ssem[b], add=True)

    def gwait(b):
        pltpu.make_async_copy(sup.at[idx[b].at[0]], rows[b], gsem[b]).wait()

    def swait(b):
        pltpu.make_async_copy(rows[b], acc.at[idx[b].at[1]], ssem[b]).wait()

    def steady(i, b):
        swait(1 - b)
        load_idx(i + 1, 1 - b)
        gstart(1 - b)
        gwait(b)
        sstart(b)

    def pipeline(nch):
        load_idx(0, 0)
        gstart(0)
        load_idx(1, 1)
        gstart(1)
        gwait(0)
        sstart(0)
        swait(0)
        load_idx(2, 0)
        gstart(0)
        gwait(1)
        sstart(1)

        def pair(j, carry):
            i = 2 + 2 * j
            steady(i, 0)
            steady(i + 1, 1)
            return carry

        lax.fori_loop(0, (nch - 4) // 2, pair, 0)

        swait(1)
        load_idx(nch - 1, 1)
        gstart(1)
        gwait(0)
        sstart(0)
        swait(0)
        gwait(1)
        sstart(1)
        swait(1)

    @pl.when(c == 0)
    def _run_core0():
        pipeline(NCH0)

    @pl.when(c == 1)
    def _run_core1():
        pipeline(NCH1)

    plsc.subcore_barrier()
    pltpu.sync_copy(acc.at[pl.ds(rbase, ROWS_PER_S)],
                    out.at[c, pl.ds(rbase, ROWS_PER_S)])


_spmm = functools.partial(
    pl.kernel,
    out_type=jax.ShapeDtypeStruct((NC, NPAD, D), jnp.float32),
    mesh=_mesh,
    scratch_types=[
        pltpu.VMEM((2, CH), jnp.int32),
        pltpu.VMEM((2, CH), jnp.int32),
        pltpu.VMEM((CH, D), jnp.float32),
        pltpu.VMEM((CH, D), jnp.float32),
        pltpu.VMEM_SHARED((NPAD, D), jnp.float32),
        pltpu.SemaphoreType.DMA,
        pltpu.SemaphoreType.DMA,
        pltpu.SemaphoreType.DMA,
        pltpu.SemaphoreType.DMA,
    ],
)(_spmm_body)


BM = 2000


def _row_spec(i_map=lambda i: (i, 0), shape=None):
    return pl.BlockSpec(shape, i_map)


def _tc1_body(x_ref, wa_ref, wb_ref, oa_ref, ob_ref):
    xv = x_ref[...]
    va = jnp.dot(xv, wa_ref[...], preferred_element_type=jnp.float32)
    vb = jnp.dot(xv, wb_ref[...], preferred_element_type=jnp.float32)
    oa_ref[...] = jnp.broadcast_to(va[None], (2,) + va.shape)
    ob_ref[...] = jnp.broadcast_to(vb[None], (2,) + vb.shape)


def _tc1(x, wa, wb):
    return pl.pallas_call(
        _tc1_body,
        grid=(N // BM,),
        in_specs=[pl.BlockSpec((BM, D), lambda i: (i, 0)),
                  pl.BlockSpec((D, D), lambda i: (0, 0)),
                  pl.BlockSpec((D, D), lambda i: (0, 0))],
        out_specs=[pl.BlockSpec((2, BM, D), lambda i: (0, i, 0))] * 2,
        out_shape=[jax.ShapeDtypeStruct((2, N, D), jnp.float32)] * 2,
    )(x, wa, wb)


def _tc2_body(p0a, p1a, ba, wa, p0b, p1b, bb, wb, oa, ob):
    ha = jnp.maximum(p0a[...] + p1a[...] + ba[...], 0.0)
    va = jnp.dot(ha, wa[...], preferred_element_type=jnp.float32)
    hb = jnp.maximum(p0b[...] + p1b[...] + bb[...], 0.0)
    vb = jnp.dot(hb, wb[...], preferred_element_type=jnp.float32)
    oa[...] = jnp.broadcast_to(va[None], (2,) + va.shape)
    ob[...] = jnp.broadcast_to(vb[None], (2,) + vb.shape)


def _tc2(p0a, p1a, ba, wa, p0b, p1b, bb, wb):
    pspec = pl.BlockSpec((BM, D), lambda i: (i, 0))
    bspec = pl.BlockSpec((1, D), lambda i: (0, 0))
    wspec = pl.BlockSpec((D, D), lambda i: (0, 0))
    return pl.pallas_call(
        _tc2_body,
        grid=(N // BM,),
        in_specs=[pspec, pspec, bspec, wspec, pspec, pspec, bspec, wspec],
        out_specs=[pl.BlockSpec((2, BM, D), lambda i: (0, i, 0))] * 2,
        out_shape=[jax.ShapeDtypeStruct((2, N, D), jnp.float32)] * 2,
    )(p0a, p1a, ba, wa, p0b, p1b, bb, wb)


def _tc3_body(p0, p1, b, w, o):
    y = p0[...] + p1[...] + b[...]
    nrm = jnp.maximum(jnp.sqrt(jnp.sum(y * y, axis=1, keepdims=True)), 1e-12)
    v = jnp.dot(y / nrm, w[...], preferred_element_type=jnp.float32)
    o[...] = jnp.broadcast_to(v[None], (2,) + v.shape)


def _tc3(p0, p1, b, w):
    pspec = pl.BlockSpec((BM, D), lambda i: (i, 0))
    return pl.pallas_call(
        _tc3_body,
        grid=(N // BM,),
        in_specs=[pspec, pspec, pl.BlockSpec((1, D), lambda i: (0, 0)),
                  pl.BlockSpec((D, D), lambda i: (0, 0))],
        out_specs=pl.BlockSpec((2, BM, D), lambda i: (0, i, 0)),
        out_shape=jax.ShapeDtypeStruct((2, N, D), jnp.float32),
    )(p0, p1, b, w)


def _tc4_body(p0, p1, b, w, o):
    h = jnp.maximum(p0[...] + p1[...] + b[...], 0.0)
    v = jnp.dot(h, w[...], preferred_element_type=jnp.float32)
    o[...] = jnp.broadcast_to(v[None], (2,) + v.shape)


def _tc4(p0, p1, b, w):
    pspec = pl.BlockSpec((BM, D), lambda i: (i, 0))
    return pl.pallas_call(
        _tc4_body,
        grid=(N // BM,),
        in_specs=[pspec, pspec, pl.BlockSpec((1, D), lambda i: (0, 0)),
                  pl.BlockSpec((D, D), lambda i: (0, 0))],
        out_specs=pl.BlockSpec((2, BM, D), lambda i: (0, i, 0)),
        out_shape=jax.ShapeDtypeStruct((2, N, D), jnp.float32),
    )(p0, p1, b, w)


def _tc5_body(x_ref, pa0, pa1, ba, q0, q1, bq, o):
    x = x_ref[...]
    ya = pa0[...] + pa1[...] + ba[...]
    na = jnp.maximum(jnp.sqrt(jnp.sum(ya * ya, axis=1, keepdims=True)), 1e-12)
    subx0 = ya / na
    yq = q0[...] + q1[...] + bq[...]
    nq = jnp.maximum(jnp.sqrt(jnp.sum(yq * yq, axis=1, keepdims=True)), 1e-12)
    subx1 = yq / nq
    s01 = (jnp.sum(x * x, axis=1, keepdims=True)
           + jnp.sum(subx0 * subx0, axis=1, keepdims=True))
    n1 = jnp.maximum(jnp.sqrt(s01), 1e-12)
    n2 = jnp.maximum(jnp.sqrt(s01 / (n1 * n1)
                              + jnp.sum(subx1 * subx1, axis=1, keepdims=True)),
                     1e-12)
    o[...] = jnp.concatenate(
        [x / (n1 * n2), subx0 / (n1 * n2), subx1 / n2], axis=1)


def _tc5(x, pa0, pa1, ba, q0, q1, bq):
    pspec = pl.BlockSpec((BM, D), lambda i: (i, 0))
    bspec = pl.BlockSpec((1, D), lambda i: (0, 0))
    return pl.pallas_call(
        _tc5_body,
        grid=(N // BM,),
        in_specs=[pspec, pspec, pspec, bspec, pspec, pspec, bspec],
        out_specs=pl.BlockSpec((BM, 3 * D), lambda i: (i, 0)),
        out_shape=jax.ShapeDtypeStruct((N, 3 * D), jnp.float32),
    )(x, pa0, pa1, ba, q0, q1, bq)


def kernel(x, edge_index, W1_00, b1_00, W2_00, b2_00, W1_10, b1_10, W2_10,
           b2_10, W1_11, b1_11, W2_11, b2_11):
    src = edge_index[0]
    dst = edge_index[1]
    srcp = jnp.concatenate([src, jnp.zeros((EPAD - E,), jnp.int32)])
    pad_dst = N + jnp.arange(EPAD - E, dtype=jnp.int32) % (NPAD - N)
    dstp = jnp.concatenate([dst, pad_dst])
    core_of_chunk = ((jnp.arange(NCH_TOT, dtype=jnp.int32) % NCHUNK)
                     >= NCH0).astype(jnp.int32)
    src_mat = srcp.reshape(NCH_TOT, CH) + core_of_chunk[:, None] * N
    eidx = jnp.stack([src_mat, dstp.reshape(NCH_TOT, CH)], axis=1)

    ba1, bb1 = b1_00.reshape(1, D), b1_10.reshape(1, D)
    ba2, bb2 = b2_00.reshape(1, D), b2_10.reshape(1, D)

    s1a, s1b = _tc1(x, W1_00, W1_10)
    a1a = _spmm(s1a.reshape(2 * N, D), eidx)
    a1b = _spmm(s1b.reshape(2 * N, D), eidx)
    s2a, s2b = _tc2(a1a[0], a1a[1], ba1, W2_00, a1b[0], a1b[1], bb1, W2_10)
    a2a = _spmm(s2a.reshape(2 * N, D), eidx)
    a2b = _spmm(s2b.reshape(2 * N, D), eidx)
    s3 = _tc3(a2b[0], a2b[1], bb2, W1_11)
    p3 = _spmm(s3.reshape(2 * N, D), eidx)
    s4 = _tc4(p3[0], p3[1], b1_11.reshape(1, D), W2_11)
    q = _spmm(s4.reshape(2 * N, D), eidx)
    return _tc5(x, a2a[0], a2a[1], ba2, q[0], q[1], b2_11.reshape(1, D))


# speedup vs baseline: 1.5688x; 1.5688x over previous
import functools

import jax
import jax.numpy as jnp
from jax import lax
from jax.experimental import pallas as pl
from jax.experimental.pallas import tpu as pltpu
from jax.experimental.pallas import tpu_sc as plsc

N = 10000
E = 320000
D = 128

NC, NS, L = 2, 16, 16
NW = NC * NS
NPAD = 10240
CH = 128
NCH0 = 116
NCH1 = 44
NCHUNK = NCH0 + NCH1
PER_S = NCHUNK * CH
EPAD = PER_S * NS
NCH_TOT = EPAD // CH
ROWS_PER_S = NPAD // NS

_mesh = plsc.VectorSubcoreMesh(core_axis_name="c", subcore_axis_name="s")


def _spmm_body(sup, eidx, out, idx0, idx1, rows0, rows1, acc,
               gsem0, gsem1, ssem0, ssem1):
    c = lax.axis_index("c")
    s = lax.axis_index("s")
    idx = (idx0, idx1)
    rows = (rows0, rows1)
    gsem = (gsem0, gsem1)
    ssem = (ssem0, ssem1)

    zeros = jnp.zeros((L,), jnp.float32)

    def zbody(r, carry):
        for k in range(D // L):
            rows0[r, pl.ds(k * L, L)] = zeros
        return carry

    lax.fori_loop(0, CH, zbody, 0)
    rbase = s * ROWS_PER_S
    for k in range(ROWS_PER_S // CH):
        pltpu.sync_copy(rows0, acc.at[pl.ds(rbase + k * CH, CH)])
    plsc.subcore_barrier()

    cbase = s * NCHUNK + c * NCH0

    def load_idx(i, b):
        pltpu.sync_copy(eidx.at[cbase + i], idx[b])

    def gstart(b):
        pltpu.async_copy(sup.at[idx[b].at[0]], rows[b], gsem[b])

    def sstart(b):
        pltpu.async_copy(rows[b], acc.at[idx[b].at[1]], ssem[b], add=True)

    def gwait(b):
        pltpu.make_async_copy(sup.at[idx[b].at[0]], rows[b], gsem[b]).wait()

    def swait(b):
        pltpu.make_async_copy(rows[b], acc.at[idx[b].at[1]], ssem[b]).wait()

    def steady(i, b):
        swait(1 - b)
        load_idx(i + 1, 1 - b)
        gstart(1 - b)
        gwait(b)
        sstart(b)

    def pipeline(nch):
        load_idx(0, 0)
        gstart(0)
        load_idx(1, 1)
        gstart(1)
        gwait(0)
        sstart(0)
        swait(0)
        load_idx(2, 0)
        gstart(0)
        gwait(1)
        sstart(1)

        def pair(j, carry):
            i = 2 + 2 * j
            steady(i, 0)
            steady(i + 1, 1)
            return carry

        lax.fori_loop(0, (nch - 4) // 2, pair, 0)

        swait(1)
        load_idx(nch - 1, 1)
        gstart(1)
        gwait(0)
        sstart(0)
        swait(0)
        gwait(1)
        sstart(1)
        swait(1)

    @pl.when(c == 0)
    def _run_core0():
        pipeline(NCH0)

    @pl.when(c == 1)
    def _run_core1():
        pipeline(NCH1)

    plsc.subcore_barrier()
    pltpu.sync_copy(acc.at[pl.ds(rbase, ROWS_PER_S)],
                    out.at[c, pl.ds(rbase, ROWS_PER_S)])


_spmm = functools.partial(
    pl.kernel,
    out_type=jax.ShapeDtypeStruct((NC, NPAD, D), jnp.float32),
    mesh=_mesh,
    scratch_types=[
        pltpu.VMEM((2, CH), jnp.int32),
        pltpu.VMEM((2, CH), jnp.int32),
        pltpu.VMEM((CH, D), jnp.float32),
        pltpu.VMEM((CH, D), jnp.float32),
        pltpu.VMEM_SHARED((NPAD, D), jnp.float32),
        pltpu.SemaphoreType.DMA,
        pltpu.SemaphoreType.DMA,
        pltpu.SemaphoreType.DMA,
        pltpu.SemaphoreType.DMA,
    ],
)(_spmm_body)


BM = 2000


def _row_spec(i_map=lambda i: (i, 0), shape=None):
    return pl.BlockSpec(shape, i_map)


def _tc1_body(x_ref, wa_ref, wb_ref, oa_ref, ob_ref):
    xv = x_ref[...]
    va = jnp.dot(xv, wa_ref[...], preferred_element_type=jnp.float32)
    vb = jnp.dot(xv, wb_ref[...], preferred_element_type=jnp.float32)
    oa_ref[...] = jnp.broadcast_to(va[None], (2,) + va.shape)
    ob_ref[...] = jnp.broadcast_to(vb[None], (2,) + vb.shape)


def _tc1(x, wa, wb):
    return pl.pallas_call(
        _tc1_body,
        grid=(N // BM,),
        in_specs=[pl.BlockSpec((BM, D), lambda i: (i, 0)),
                  pl.BlockSpec((D, D), lambda i: (0, 0)),
                  pl.BlockSpec((D, D), lambda i: (0, 0))],
        out_specs=[pl.BlockSpec((2, BM, D), lambda i: (0, i, 0))] * 2,
        out_shape=[jax.ShapeDtypeStruct((2, N, D), jnp.float32)] * 2,
    )(x, wa, wb)


def _tc2_body(pa, ba, wa, pb, bb, wb, oa, ob):
    ha = jnp.maximum(pa[0] + pa[1] + ba[...], 0.0)
    va = jnp.dot(ha, wa[...], preferred_element_type=jnp.float32)
    hb = jnp.maximum(pb[0] + pb[1] + bb[...], 0.0)
    vb = jnp.dot(hb, wb[...], preferred_element_type=jnp.float32)
    oa[...] = jnp.broadcast_to(va[None], (2,) + va.shape)
    ob[...] = jnp.broadcast_to(vb[None], (2,) + vb.shape)


def _tc2(pa, ba, wa, pb, bb, wb):
    pspec = pl.BlockSpec((2, BM, D), lambda i: (0, i, 0))
    bspec = pl.BlockSpec((1, D), lambda i: (0, 0))
    wspec = pl.BlockSpec((D, D), lambda i: (0, 0))
    return pl.pallas_call(
        _tc2_body,
        grid=(N // BM,),
        in_specs=[pspec, bspec, wspec, pspec, bspec, wspec],
        out_specs=[pl.BlockSpec((2, BM, D), lambda i: (0, i, 0))] * 2,
        out_shape=[jax.ShapeDtypeStruct((2, N, D), jnp.float32)] * 2,
    )(pa, ba, wa, pb, bb, wb)


def _tc3_body(p, b, w, o):
    y = p[0] + p[1] + b[...]
    nrm = jnp.maximum(jnp.sqrt(jnp.sum(y * y, axis=1, keepdims=True)), 1e-12)
    v = jnp.dot(y / nrm, w[...], preferred_element_type=jnp.float32)
    o[...] = jnp.broadcast_to(v[None], (2,) + v.shape)


def _tc3(p, b, w):
    pspec = pl.BlockSpec((2, BM, D), lambda i: (0, i, 0))
    return pl.pallas_call(
        _tc3_body,
        grid=(N // BM,),
        in_specs=[pspec, pl.BlockSpec((1, D), lambda i: (0, 0)),
                  pl.BlockSpec((D, D), lambda i: (0, 0))],
        out_specs=pl.BlockSpec((2, BM, D), lambda i: (0, i, 0)),
        out_shape=jax.ShapeDtypeStruct((2, N, D), jnp.float32),
    )(p, b, w)


def _tc4_body(p, b, w, o):
    h = jnp.maximum(p[0] + p[1] + b[...], 0.0)
    v = jnp.dot(h, w[...], preferred_element_type=jnp.float32)
    o[...] = jnp.broadcast_to(v[None], (2,) + v.shape)


def _tc4(p, b, w):
    pspec = pl.BlockSpec((2, BM, D), lambda i: (0, i, 0))
    return pl.pallas_call(
        _tc4_body,
        grid=(N // BM,),
        in_specs=[pspec, pl.BlockSpec((1, D), lambda i: (0, 0)),
                  pl.BlockSpec((D, D), lambda i: (0, 0))],
        out_specs=pl.BlockSpec((2, BM, D), lambda i: (0, i, 0)),
        out_shape=jax.ShapeDtypeStruct((2, N, D), jnp.float32),
    )(p, b, w)


def _tc5_body(x_ref, pa, ba, q, bq, o):
    x = x_ref[...]
    ya = pa[0] + pa[1] + ba[...]
    na = jnp.maximum(jnp.sqrt(jnp.sum(ya * ya, axis=1, keepdims=True)), 1e-12)
    subx0 = ya / na
    yq = q[0] + q[1] + bq[...]
    nq = jnp.maximum(jnp.sqrt(jnp.sum(yq * yq, axis=1, keepdims=True)), 1e-12)
    subx1 = yq / nq
    s01 = (jnp.sum(x * x, axis=1, keepdims=True)
           + jnp.sum(subx0 * subx0, axis=1, keepdims=True))
    n1 = jnp.maximum(jnp.sqrt(s01), 1e-12)
    n2 = jnp.maximum(jnp.sqrt(s01 / (n1 * n1)
                              + jnp.sum(subx1 * subx1, axis=1, keepdims=True)),
                     1e-12)
    o[...] = jnp.concatenate(
        [x / (n1 * n2), subx0 / (n1 * n2), subx1 / n2], axis=1)


def _tc5(x, pa, ba, q, bq):
    xspec = pl.BlockSpec((BM, D), lambda i: (i, 0))
    pspec = pl.BlockSpec((2, BM, D), lambda i: (0, i, 0))
    bspec = pl.BlockSpec((1, D), lambda i: (0, 0))
    return pl.pallas_call(
        _tc5_body,
        grid=(N // BM,),
        in_specs=[xspec, pspec, bspec, pspec, bspec],
        out_specs=pl.BlockSpec((BM, 3 * D), lambda i: (i, 0)),
        out_shape=jax.ShapeDtypeStruct((N, 3 * D), jnp.float32),
    )(x, pa, ba, q, bq)


def kernel(x, edge_index, W1_00, b1_00, W2_00, b2_00, W1_10, b1_10, W2_10,
           b2_10, W1_11, b1_11, W2_11, b2_11):
    src = edge_index[0]
    dst = edge_index[1]
    srcp = jnp.concatenate([src, jnp.zeros((EPAD - E,), jnp.int32)])
    pad_dst = N + jnp.arange(EPAD - E, dtype=jnp.int32) % (NPAD - N)
    dstp = jnp.concatenate([dst, pad_dst])
    core_of_chunk = ((jnp.arange(NCH_TOT, dtype=jnp.int32) % NCHUNK)
                     >= NCH0).astype(jnp.int32)
    src_mat = srcp.reshape(NCH_TOT, CH) + core_of_chunk[:, None] * N
    eidx = jnp.stack([src_mat, dstp.reshape(NCH_TOT, CH)], axis=1)

    ba1, bb1 = b1_00.reshape(1, D), b1_10.reshape(1, D)
    ba2, bb2 = b2_00.reshape(1, D), b2_10.reshape(1, D)

    s1a, s1b = _tc1(x, W1_00, W1_10)
    a1a = _spmm(s1a.reshape(2 * N, D), eidx)
    a1b = _spmm(s1b.reshape(2 * N, D), eidx)
    s2a, s2b = _tc2(a1a, ba1, W2_00, a1b, bb1, W2_10)
    a2a = _spmm(s2a.reshape(2 * N, D), eidx)
    a2b = _spmm(s2b.reshape(2 * N, D), eidx)
    s3 = _tc3(a2b, bb2, W1_11)
    p3 = _spmm(s3.reshape(2 * N, D), eidx)
    s4 = _tc4(p3, b1_11.reshape(1, D), W2_11)
    q = _spmm(s4.reshape(2 * N, D), eidx)
    return _tc5(x, a2a, ba2, q, b2_11.reshape(1, D))


# speedup vs baseline: 1.6981x; 1.0824x over previous
import functools

import jax
import jax.numpy as jnp
from jax import lax
from jax.experimental import pallas as pl
from jax.experimental.pallas import tpu as pltpu
from jax.experimental.pallas import tpu_sc as plsc

N = 10000
E = 320000
D = 128

NC, NS, L = 2, 16, 16
NW = NC * NS
NPAD = 10240
CH = 128
NCH0 = 120
NCH1 = 40
NCHUNK = NCH0 + NCH1
PER_S = NCHUNK * CH
EPAD = PER_S * NS
NCH_TOT = EPAD // CH
ROWS_PER_S = NPAD // NS

_mesh = plsc.VectorSubcoreMesh(core_axis_name="c", subcore_axis_name="s")


def _spmm_body(sup, eidx, out, idx0, idx1, rows0, rows1, acc,
               gsem0, gsem1, ssem0, ssem1):
    c = lax.axis_index("c")
    s = lax.axis_index("s")
    idx = (idx0, idx1)
    rows = (rows0, rows1)
    gsem = (gsem0, gsem1)
    ssem = (ssem0, ssem1)

    zeros = jnp.zeros((L,), jnp.float32)

    def zbody(r, carry):
        for k in range(D // L):
            rows0[r, pl.ds(k * L, L)] = zeros
        return carry

    lax.fori_loop(0, CH, zbody, 0)
    rbase = s * ROWS_PER_S
    for k in range(ROWS_PER_S // CH):
        pltpu.sync_copy(rows0, acc.at[pl.ds(rbase + k * CH, CH)])
    plsc.subcore_barrier()

    cbase = s * NCHUNK + c * NCH0

    def load_idx(i, b):
        pltpu.sync_copy(eidx.at[cbase + i], idx[b])

    def gstart(b):
        pltpu.async_copy(sup.at[idx[b].at[0]], rows[b], gsem[b])

    def sstart(b):
        pltpu.async_copy(rows[b], acc.at[idx[b].at[1]], ssem[b], add=True)

    def gwait(b):
        pltpu.make_async_copy(sup.at[idx[b].at[0]], rows[b], gsem[b]).wait()

    def swait(b):
        pltpu.make_async_copy(rows[b], acc.at[idx[b].at[1]], ssem[b]).wait()

    def steady(i, b):
        swait(1 - b)
        load_idx(i + 1, 1 - b)
        gstart(1 - b)
        gwait(b)
        sstart(b)

    def pipeline(nch):
        load_idx(0, 0)
        gstart(0)
        load_idx(1, 1)
        gstart(1)
        gwait(0)
        sstart(0)
        swait(0)
        load_idx(2, 0)
        gstart(0)
        gwait(1)
        sstart(1)

        def pair(j, carry):
            i = 2 + 2 * j
            steady(i, 0)
            steady(i + 1, 1)
            return carry

        lax.fori_loop(0, (nch - 4) // 2, pair, 0)

        swait(1)
        load_idx(nch - 1, 1)
        gstart(1)
        gwait(0)
        sstart(0)
        swait(0)
        gwait(1)
        sstart(1)
        swait(1)

    @pl.when(c == 0)
    def _run_core0():
        pipeline(NCH0)

    @pl.when(c == 1)
    def _run_core1():
        pipeline(NCH1)

    plsc.subcore_barrier()
    pltpu.sync_copy(acc.at[pl.ds(rbase, ROWS_PER_S)],
                    out.at[c, pl.ds(rbase, ROWS_PER_S)])


_spmm = functools.partial(
    pl.kernel,
    out_type=jax.ShapeDtypeStruct((NC, NPAD, D), jnp.float32),
    mesh=_mesh,
    scratch_types=[
        pltpu.VMEM((2, CH), jnp.int32),
        pltpu.VMEM((2, CH), jnp.int32),
        pltpu.VMEM((CH, D), jnp.float32),
        pltpu.VMEM((CH, D), jnp.float32),
        pltpu.VMEM_SHARED((NPAD, D), jnp.float32),
        pltpu.SemaphoreType.DMA,
        pltpu.SemaphoreType.DMA,
        pltpu.SemaphoreType.DMA,
        pltpu.SemaphoreType.DMA,
    ],
)(_spmm_body)


BM = 2000


def _row_spec(i_map=lambda i: (i, 0), shape=None):
    return pl.BlockSpec(shape, i_map)


def _tc1_body(x_ref, wa_ref, wb_ref, oa_ref, ob_ref):
    xv = x_ref[...]
    va = jnp.dot(xv, wa_ref[...], preferred_element_type=jnp.float32)
    vb = jnp.dot(xv, wb_ref[...], preferred_element_type=jnp.float32)
    oa_ref[...] = jnp.broadcast_to(va[None], (2,) + va.shape)
    ob_ref[...] = jnp.broadcast_to(vb[None], (2,) + vb.shape)


def _tc1(x, wa, wb):
    return pl.pallas_call(
        _tc1_body,
        grid=(N // BM,),
        in_specs=[pl.BlockSpec((BM, D), lambda i: (i, 0)),
                  pl.BlockSpec((D, D), lambda i: (0, 0)),
                  pl.BlockSpec((D, D), lambda i: (0, 0))],
        out_specs=[pl.BlockSpec((2, BM, D), lambda i: (0, i, 0))] * 2,
        out_shape=[jax.ShapeDtypeStruct((2, N, D), jnp.float32)] * 2,
    )(x, wa, wb)


def _tc2_body(pa, ba, wa, pb, bb, wb, oa, ob):
    ha = jnp.maximum(pa[0] + pa[1] + ba[...], 0.0)
    va = jnp.dot(ha, wa[...], preferred_element_type=jnp.float32)
    hb = jnp.maximum(pb[0] + pb[1] + bb[...], 0.0)
    vb = jnp.dot(hb, wb[...], preferred_element_type=jnp.float32)
    oa[...] = jnp.broadcast_to(va[None], (2,) + va.shape)
    ob[...] = jnp.broadcast_to(vb[None], (2,) + vb.shape)


def _tc2(pa, ba, wa, pb, bb, wb):
    pspec = pl.BlockSpec((2, BM, D), lambda i: (0, i, 0))
    bspec = pl.BlockSpec((1, D), lambda i: (0, 0))
    wspec = pl.BlockSpec((D, D), lambda i: (0, 0))
    return pl.pallas_call(
        _tc2_body,
        grid=(N // BM,),
        in_specs=[pspec, bspec, wspec, pspec, bspec, wspec],
        out_specs=[pl.BlockSpec((2, BM, D), lambda i: (0, i, 0))] * 2,
        out_shape=[jax.ShapeDtypeStruct((2, N, D), jnp.float32)] * 2,
    )(pa, ba, wa, pb, bb, wb)


def _tc3_body(p, b, w, o):
    y = p[0] + p[1] + b[...]
    nrm = jnp.maximum(jnp.sqrt(jnp.sum(y * y, axis=1, keepdims=True)), 1e-12)
    v = jnp.dot(y / nrm, w[...], preferred_element_type=jnp.float32)
    o[...] = jnp.broadcast_to(v[None], (2,) + v.shape)


def _tc3(p, b, w):
    pspec = pl.BlockSpec((2, BM, D), lambda i: (0, i, 0))
    return pl.pallas_call(
        _tc3_body,
        grid=(N // BM,),
        in_specs=[pspec, pl.BlockSpec((1, D), lambda i: (0, 0)),
                  pl.BlockSpec((D, D), lambda i: (0, 0))],
        out_specs=pl.BlockSpec((2, BM, D), lambda i: (0, i, 0)),
        out_shape=jax.ShapeDtypeStruct((2, N, D), jnp.float32),
    )(p, b, w)


def _tc4_body(p, b, w, o):
    h = jnp.maximum(p[0] + p[1] + b[...], 0.0)
    v = jnp.dot(h, w[...], preferred_element_type=jnp.float32)
    o[...] = jnp.broadcast_to(v[None], (2,) + v.shape)


def _tc4(p, b, w):
    pspec = pl.BlockSpec((2, BM, D), lambda i: (0, i, 0))
    return pl.pallas_call(
        _tc4_body,
        grid=(N // BM,),
        in_specs=[pspec, pl.BlockSpec((1, D), lambda i: (0, 0)),
                  pl.BlockSpec((D, D), lambda i: (0, 0))],
        out_specs=pl.BlockSpec((2, BM, D), lambda i: (0, i, 0)),
        out_shape=jax.ShapeDtypeStruct((2, N, D), jnp.float32),
    )(p, b, w)


def _tc5_body(x_ref, pa, ba, q, bq, o):
    x = x_ref[...]
    ya = pa[0] + pa[1] + ba[...]
    na = jnp.maximum(jnp.sqrt(jnp.sum(ya * ya, axis=1, keepdims=True)), 1e-12)
    subx0 = ya / na
    yq = q[0] + q[1] + bq[...]
    nq = jnp.maximum(jnp.sqrt(jnp.sum(yq * yq, axis=1, keepdims=True)), 1e-12)
    subx1 = yq / nq
    s01 = (jnp.sum(x * x, axis=1, keepdims=True)
           + jnp.sum(subx0 * subx0, axis=1, keepdims=True))
    n1 = jnp.maximum(jnp.sqrt(s01), 1e-12)
    n2 = jnp.maximum(jnp.sqrt(s01 / (n1 * n1)
                              + jnp.sum(subx1 * subx1, axis=1, keepdims=True)),
                     1e-12)
    o[...] = jnp.concatenate(
        [x / (n1 * n2), subx0 / (n1 * n2), subx1 / n2], axis=1)


def _tc5(x, pa, ba, q, bq):
    xspec = pl.BlockSpec((BM, D), lambda i: (i, 0))
    pspec = pl.BlockSpec((2, BM, D), lambda i: (0, i, 0))
    bspec = pl.BlockSpec((1, D), lambda i: (0, 0))
    return pl.pallas_call(
        _tc5_body,
        grid=(N // BM,),
        in_specs=[xspec, pspec, bspec, pspec, bspec],
        out_specs=pl.BlockSpec((BM, 3 * D), lambda i: (i, 0)),
        out_shape=jax.ShapeDtypeStruct((N, 3 * D), jnp.float32),
    )(x, pa, ba, q, bq)


def kernel(x, edge_index, W1_00, b1_00, W2_00, b2_00, W1_10, b1_10, W2_10,
           b2_10, W1_11, b1_11, W2_11, b2_11):
    src = edge_index[0]
    dst = edge_index[1]
    srcp = jnp.concatenate([src, jnp.zeros((EPAD - E,), jnp.int32)])
    pad_dst = N + jnp.arange(EPAD - E, dtype=jnp.int32) % (NPAD - N)
    dstp = jnp.concatenate([dst, pad_dst])
    core_of_chunk = ((jnp.arange(NCH_TOT, dtype=jnp.int32) % NCHUNK)
                     >= NCH0).astype(jnp.int32)
    src_mat = srcp.reshape(NCH_TOT, CH) + core_of_chunk[:, None] * N
    eidx = jnp.stack([src_mat, dstp.reshape(NCH_TOT, CH)], axis=1)

    ba1, bb1 = b1_00.reshape(1, D), b1_10.reshape(1, D)
    ba2, bb2 = b2_00.reshape(1, D), b2_10.reshape(1, D)

    s1a, s1b = _tc1(x, W1_00, W1_10)
    a1a = _spmm(s1a.reshape(2 * N, D), eidx)
    a1b = _spmm(s1b.reshape(2 * N, D), eidx)
    s2a, s2b = _tc2(a1a, ba1, W2_00, a1b, bb1, W2_10)
    a2a = _spmm(s2a.reshape(2 * N, D), eidx)
    a2b = _spmm(s2b.reshape(2 * N, D), eidx)
    s3 = _tc3(a2b, bb2, W1_11)
    p3 = _spmm(s3.reshape(2 * N, D), eidx)
    s4 = _tc4(p3, b1_11.reshape(1, D), W2_11)
    q = _spmm(s4.reshape(2 * N, D), eidx)
    return _tc5(x, a2a, ba2, q, b2_11.reshape(1, D))
